# baseline (speedup 1.0000x reference)
import functools

import jax
import jax.numpy as jnp
from jax import lax
from jax.experimental import pallas as pl
from jax.experimental.pallas import tpu as pltpu
from jax.experimental.pallas import tpu_sc as plsc

DIM = 64

NC = 2
NS = 16
NW = NC * NS

IDXW = 128
CHUNK = 1024
GATHERS = CHUNK // IDXW


@functools.partial(jax.jit, static_argnums=(2,))
def _embed(idx2d, table, b_per_w):
    B = idx2d.shape[0] * IDXW
    n_chunks = b_per_w // CHUNK

    mesh = plsc.VectorSubcoreMesh(
        core_axis_name="c", subcore_axis_name="s", num_cores=NC,
        num_subcores=NS)

    @functools.partial(
        pl.kernel,
        out_type=jax.ShapeDtypeStruct((B, DIM), jnp.float32),
        mesh=mesh,
        compiler_params=pltpu.CompilerParams(use_tc_tiling_on_sc=False),
        scratch_types=[
            pltpu.VMEM((GATHERS, IDXW), jnp.int32),
            pltpu.VMEM((CHUNK, DIM), jnp.float32),
            pltpu.SemaphoreType.DMA,
        ],
    )
    def k(idx_hbm, table_hbm, out_hbm, idx_v, rows_v, sem):
        wid = lax.axis_index("s") * NC + lax.axis_index("c")
        base = wid * b_per_w

        def chunk_body(i, carry):
            off = pl.multiple_of(base + i * CHUNK, CHUNK)
            pltpu.sync_copy(
                idx_hbm.at[pl.ds(pl.multiple_of(off // IDXW, 8), GATHERS)],
                idx_v)
            copies = [
                pltpu.async_copy(
                    table_hbm.at[idx_v.at[j]],
                    rows_v.at[pl.ds(j * IDXW, IDXW)],
                    sem,
                )
                for j in range(GATHERS)
            ]
            for c in copies:
                c.wait()
            pltpu.sync_copy(rows_v, out_hbm.at[pl.ds(off, CHUNK)])
            return carry

        lax.fori_loop(0, n_chunks, chunk_body, 0)

    return k(idx2d, table)


def kernel(x, table):
    s0, s1 = x.shape
    b = s0 * s1
    b_per_w = b // NW
    idx2d = x.reshape(-1).astype(jnp.int32).reshape(b // IDXW, IDXW)
    out = _embed(idx2d, table, b_per_w)
    return out.reshape(s0, s1, DIM)


# speedup vs baseline: 1.0175x; 1.0175x over previous
import functools

import jax
import jax.numpy as jnp
from jax import lax
from jax.experimental import pallas as pl
from jax.experimental.pallas import tpu as pltpu
from jax.experimental.pallas import tpu_sc as plsc

DIM = 64

NC = 2
NS = 16
NW = NC * NS

IDXW = 128
CHUNK = 512
GATHERS = CHUNK // IDXW
NBUF = 2


@functools.partial(jax.jit, static_argnums=(2,))
def _embed(idx2d, table, b_per_w):
    B = idx2d.shape[0] * IDXW
    n_chunks = b_per_w // CHUNK
    rows_per_w = b_per_w // IDXW

    mesh = plsc.VectorSubcoreMesh(
        core_axis_name="c", subcore_axis_name="s", num_cores=NC,
        num_subcores=NS)

    @functools.partial(
        pl.kernel,
        out_type=jax.ShapeDtypeStruct((B, DIM), jnp.float32),
        mesh=mesh,
        compiler_params=pltpu.CompilerParams(use_tc_tiling_on_sc=False),
        scratch_types=[
            pltpu.VMEM((rows_per_w, IDXW), jnp.int32),
            pltpu.VMEM((NBUF, CHUNK, DIM), jnp.float32),
            pltpu.SemaphoreType.DMA,
            pltpu.SemaphoreType.DMA,
            pltpu.SemaphoreType.DMA,
            pltpu.SemaphoreType.DMA,
        ],
    )
    def k(idx_hbm, table_hbm, out_hbm, idx_v, rows_v, g0, g1, o0, o1):
        gsems = (g0, g1)
        osems = (o0, o1)
        wid = lax.axis_index("s") * NC + lax.axis_index("c")
        base = wid * b_per_w

        pltpu.sync_copy(
            idx_hbm.at[pl.ds(pl.multiple_of(base // IDXW, 8), rows_per_w)],
            idx_v)

        def out_off(i):
            return pl.multiple_of(base + i * CHUNK, CHUNK)

        def fire_gathers(i, b):
            for j in range(GATHERS):
                pltpu.async_copy(
                    table_hbm.at[idx_v.at[i * GATHERS + j]],
                    rows_v.at[b, pl.ds(j * IDXW, IDXW)],
                    gsems[b])

        def wait_gathers(b):
            pltpu.make_async_copy(
                table_hbm.at[pl.ds(0, CHUNK)], rows_v.at[b], gsems[b]).wait()

        def fire_out(i, b):
            pltpu.async_copy(
                rows_v.at[b], out_hbm.at[pl.ds(out_off(i), CHUNK)], osems[b])

        def wait_out(b):
            pltpu.make_async_copy(
                rows_v.at[b], out_hbm.at[pl.ds(0, CHUNK)], osems[b]).wait()

        fire_gathers(0, 0)
        fire_gathers(1, 1)
        wait_gathers(0)
        fire_out(0, 0)

        @pl.loop(NBUF, n_chunks, step=NBUF)
        def _(i0):
            for d in range(NBUF):
                i = i0 + d
                b = d
                ob = 1 - b
                wait_out(b)
                fire_gathers(i, b)
                wait_gathers(ob)
                fire_out(i - 1, ob)

        wait_gathers(1)
        fire_out(n_chunks - 1, 1)
        wait_out(0)
        wait_out(1)

    return k(idx2d, table)


def kernel(x, table):
    s0, s1 = x.shape
    b = s0 * s1
    b_per_w = b // NW
    idx2d = x.reshape(-1).astype(jnp.int32).reshape(b // IDXW, IDXW)
    out = _embed(idx2d, table, b_per_w)
    return out.reshape(s0, s1, DIM)
